# baseline (speedup 1.0000x reference)
import jax
import jax.numpy as jnp
from jax import lax
from jax.experimental import pallas as pl
from jax.experimental.pallas import tpu as pltpu
from jax.experimental.pallas import tpu_sc as plsc

B = 16384
D = 512
SEG = 64
NSEG = 8
V = 100000

NC = 2
NS = 16
NW = NC * NS
L = 16

B_PER_W = B // NW
CHUNK = 64
N_CHUNKS = B_PER_W // CHUNK
N_PAIRS = N_CHUNKS // 2
NEWTON_ITERS = 1


def _rsqrt(v):
    i = plsc.bitcast(v, jnp.int32)
    i = jnp.int32(0x5F3759DF) - lax.shift_right_logical(i, 1)
    y = plsc.bitcast(i, jnp.float32)
    half = v * jnp.float32(0.5)
    for _ in range(NEWTON_ITERS):
        y = y * (jnp.float32(1.5) - half * y * y)
    return y


def _body(idx_hbm, table_hbm, gamma_hbm, beta_hbm, out_hbm,
          idx_v, buf0, buf1, gamma_v, beta_v,
          in0, in1, out0, out1):
    wid = lax.axis_index("c") * NS + lax.axis_index("s")
    base = wid * B_PER_W

    pltpu.sync_copy(idx_hbm.at[pl.ds(base, B_PER_W)], idx_v)
    pltpu.sync_copy(gamma_hbm, gamma_v)
    pltpu.sync_copy(beta_hbm, beta_v)

    gv = [gamma_v[pl.ds(16 * k, 16)] for k in range(4)]
    bv = [beta_v[pl.ds(16 * k, 16)] for k in range(4)]
    inv_seg = jnp.float32(1.0 / SEG)
    eps = jnp.float32(1e-5)

    def start_gather(c, buf, sem):
        pltpu.make_async_copy(
            table_hbm.at[idx_v.at[pl.ds(c * CHUNK, CHUNK)]], buf, sem
        ).start()

    def wait_gather(c, buf, sem):
        pltpu.make_async_copy(
            table_hbm.at[idx_v.at[pl.ds(c * CHUNK, CHUNK)]], buf, sem
        ).wait()

    def start_out(c, buf, sem):
        pltpu.make_async_copy(
            buf, out_hbm.at[pl.ds(base + c * CHUNK, CHUNK)], sem
        ).start()

    def wait_out(c, buf, sem):
        pltpu.make_async_copy(
            buf, out_hbm.at[pl.ds(base + c * CHUNK, CHUNK)], sem
        ).wait()

    def layernorm_chunk(buf):
        def row_step(r2, _):
            for r in (2 * r2, 2 * r2 + 1):
                _ln_row(buf, r)
            return 0

        def _ln_row(buf, r):
            for s in range(NSEG):
                xs = [buf[r, pl.ds(s * SEG + 16 * k, 16)] for k in range(4)]
                ssum = jnp.sum(xs[0] + xs[1] + xs[2] + xs[3])
                qsum = jnp.sum(xs[0] * xs[0] + xs[1] * xs[1]
                               + xs[2] * xs[2] + xs[3] * xs[3])
                mean_s = ssum * inv_seg
                var_s = qsum * inv_seg - mean_s * mean_s
                mean = jnp.full((L,), mean_s)
                rstd = _rsqrt(jnp.full((L,), var_s + eps))
                for k in range(4):
                    buf[r, pl.ds(s * SEG + 16 * k, 16)] = (
                        (xs[k] - mean) * rstd * gv[k] + bv[k])

        lax.fori_loop(0, CHUNK // 2, row_step, 0)

    start_gather(0, buf0, in0)

    def pair_step(t, _):
        c0 = 2 * t
        c1 = c0 + 1

        @pl.when(t > 0)
        def _():
            wait_out(c0 - 1, buf1, out1)

        start_gather(c1, buf1, in1)
        wait_gather(c0, buf0, in0)
        layernorm_chunk(buf0)
        start_out(c0, buf0, out0)

        wait_gather(c1, buf1, in1)
        layernorm_chunk(buf1)
        start_out(c1, buf1, out1)

        @pl.when(t < N_PAIRS - 1)
        def _():
            wait_out(c0, buf0, out0)
            start_gather(c0 + 2, buf0, in0)

        return 0

    lax.fori_loop(0, N_PAIRS, pair_step, 0)
    wait_out(N_CHUNKS - 2, buf0, out0)
    wait_out(N_CHUNKS - 1, buf1, out1)


@jax.jit
def _encode(prompt_idx, table, gamma, beta):
    mesh = plsc.VectorSubcoreMesh(core_axis_name="c", subcore_axis_name="s")
    run = pl.kernel(
        _body,
        out_type=jax.ShapeDtypeStruct((B, D), jnp.float32),
        mesh=mesh,
        compiler_params=pltpu.CompilerParams(needs_layout_passes=False),
        scratch_types=[
            pltpu.VMEM((B_PER_W,), jnp.int32),
            pltpu.VMEM((CHUNK, D), jnp.float32),
            pltpu.VMEM((CHUNK, D), jnp.float32),
            pltpu.VMEM((SEG,), jnp.float32),
            pltpu.VMEM((SEG,), jnp.float32),
            pltpu.SemaphoreType.DMA,
            pltpu.SemaphoreType.DMA,
            pltpu.SemaphoreType.DMA,
            pltpu.SemaphoreType.DMA,
        ],
    )
    return run(prompt_idx, table, gamma, beta)


def kernel(prompt_idx, table, gamma, beta):
    out = _encode(prompt_idx, table, gamma, beta)
    return out.reshape(B, NSEG, SEG)


# speedup vs baseline: 1.3111x; 1.3111x over previous
import jax
import jax.numpy as jnp
from jax import lax
from jax.experimental import pallas as pl
from jax.experimental.pallas import tpu as pltpu
from jax.experimental.pallas import tpu_sc as plsc

B = 16384
D = 512
SEG = 64
NSEG = 8
V = 100000

NC = 2
NS = 16
NW = NC * NS
L = 16

B_PER_W = B // NW
CHUNK = 64
N_CHUNKS = B_PER_W // CHUNK
N_PAIRS = N_CHUNKS // 2
NEWTON_ITERS = 1


def _rsqrt(v):
    i = plsc.bitcast(v, jnp.int32)
    i = jnp.int32(0x5F3759DF) - lax.shift_right_logical(i, 1)
    y = plsc.bitcast(i, jnp.float32)
    half = v * jnp.float32(0.5)
    for _ in range(NEWTON_ITERS):
        y = y * (jnp.float32(1.5) - half * y * y)
    return y


def _body(idx_hbm, table_hbm, gamma_hbm, beta_hbm, out_hbm,
          idx_v, buf0, buf1, gamma_v, beta_v,
          in0, in1, out0, out1):
    wid = lax.axis_index("c") * NS + lax.axis_index("s")
    base = wid * B_PER_W

    pltpu.sync_copy(idx_hbm.at[pl.ds(base, B_PER_W)], idx_v)
    pltpu.sync_copy(gamma_hbm, gamma_v)
    pltpu.sync_copy(beta_hbm, beta_v)

    gv = [gamma_v[pl.ds(16 * k, 16)] for k in range(4)]
    bv = [beta_v[pl.ds(16 * k, 16)] for k in range(4)]
    inv_seg = jnp.float32(1.0 / SEG)
    eps = jnp.float32(1e-5)

    def start_gather(c, buf, sem):
        pltpu.make_async_copy(
            table_hbm.at[idx_v.at[pl.ds(c * CHUNK, CHUNK)]], buf, sem
        ).start()

    def wait_gather(c, buf, sem):
        pltpu.make_async_copy(
            table_hbm.at[idx_v.at[pl.ds(c * CHUNK, CHUNK)]], buf, sem
        ).wait()

    def start_out(c, buf, sem):
        pltpu.make_async_copy(
            buf, out_hbm.at[pl.ds(base + c * CHUNK, CHUNK)], sem
        ).start()

    def wait_out(c, buf, sem):
        pltpu.make_async_copy(
            buf, out_hbm.at[pl.ds(base + c * CHUNK, CHUNK)], sem
        ).wait()

    def layernorm_chunk(buf):
        def row_step(r, _):
            for s in range(NSEG):
                xs = [buf[r, pl.ds(s * SEG + 16 * k, 16)] for k in range(4)]
                ssum = jnp.sum(xs[0] + xs[1] + xs[2] + xs[3])
                qsum = jnp.sum(xs[0] * xs[0] + xs[1] * xs[1]
                               + xs[2] * xs[2] + xs[3] * xs[3])
                mean_s = ssum * inv_seg
                var_s = qsum * inv_seg - mean_s * mean_s
                mean = jnp.full((L,), mean_s)
                rstd = _rsqrt(jnp.full((L,), var_s + eps))
                for k in range(4):
                    buf[r, pl.ds(s * SEG + 16 * k, 16)] = (
                        (xs[k] - mean) * rstd * gv[k] + bv[k])
            return 0

        lax.fori_loop(0, CHUNK, row_step, 0)

    start_gather(0, buf0, in0)

    def pair_step(t, _):
        c0 = 2 * t
        c1 = c0 + 1

        @pl.when(t > 0)
        def _():
            wait_out(c0 - 1, buf1, out1)

        start_gather(c1, buf1, in1)
        wait_gather(c0, buf0, in0)
        layernorm_chunk(buf0)
        start_out(c0, buf0, out0)

        wait_gather(c1, buf1, in1)
        layernorm_chunk(buf1)
        start_out(c1, buf1, out1)

        @pl.when(t < N_PAIRS - 1)
        def _():
            wait_out(c0, buf0, out0)
            start_gather(c0 + 2, buf0, in0)

        return 0

    lax.fori_loop(0, N_PAIRS, pair_step, 0)
    wait_out(N_CHUNKS - 2, buf0, out0)
    wait_out(N_CHUNKS - 1, buf1, out1)


@jax.jit
def _encode(prompt_idx, table, gamma, beta):
    mesh = plsc.VectorSubcoreMesh(core_axis_name="c", subcore_axis_name="s")
    run = pl.kernel(
        _body,
        out_type=jax.ShapeDtypeStruct((B, D), jnp.float32),
        mesh=mesh,
        compiler_params=pltpu.CompilerParams(needs_layout_passes=False),
        scratch_types=[
            pltpu.VMEM((B_PER_W,), jnp.int32),
            pltpu.VMEM((CHUNK, D), jnp.float32),
            pltpu.VMEM((CHUNK, D), jnp.float32),
            pltpu.VMEM((SEG,), jnp.float32),
            pltpu.VMEM((SEG,), jnp.float32),
            pltpu.SemaphoreType.DMA,
            pltpu.SemaphoreType.DMA,
            pltpu.SemaphoreType.DMA,
            pltpu.SemaphoreType.DMA,
        ],
    )
    return run(prompt_idx, table, gamma, beta)


def kernel(prompt_idx, table, gamma, beta):
    out = _encode(prompt_idx, table, gamma, beta)
    return out.reshape(B, NSEG, SEG)
